# baseline (speedup 1.0000x reference)
import functools

import jax
import jax.numpy as jnp
from jax import lax
from jax.experimental import pallas as pl
from jax.experimental.pallas import tpu as pltpu
from jax.experimental.pallas import tpu_sc as plsc

_B = 16384
_D = 512
_C = 1000
_CPAD = 1024

_BSC = 8192
_BTC = _B - _BSC
_RB = 256

_NC = 2
_NS = 16
_NW = _NC * _NS
_BPW = _BSC // _NW
_CH = 32
_NCHUNK = _BPW // _CH
_NBUF = 2


def _body(feats_hbm, labels_hbm, centers_hbm, out_hbm,
          idx_all, fb0, fb1, gb0, gb1, acc_v, tab_sh,
          sf0, sf1, sg0, sg1, sl):
    cid = lax.axis_index("c")
    sid = lax.axis_index("s")
    wid = sid * _NC + cid
    base = wid * _BPW

    fbs = (fb0, fb1)
    gbs = (gb0, gb1)
    sfs = (sf0, sf1)
    sgs = (sg0, sg1)

    @pl.when(sid < 8)
    def _():
        o = sid * (_C // 8) * _D
        pltpu.sync_copy(centers_hbm.at[pl.ds(o, (_C // 8) * _D)],
                        tab_sh.at[pl.ds(o, (_C // 8) * _D)])

    lcps = [
        pltpu.async_copy(labels_hbm.at[pl.ds(base + k * _CH, _CH)],
                         idx_all.at[k], sl)
        for k in range(_NCHUNK)
    ]
    for cp in lcps:
        cp.wait()
    plsc.subcore_barrier()

    def issue(k, b):
        pltpu.async_copy(feats_hbm.at[pl.ds(base + k * _CH, _CH)],
                         fbs[b], sfs[b])
        for j in range(_CH // 16):
            ixv = idx_all[k, pl.ds(j * 16, 16)]
            for l in range(16):
                off = pl.multiple_of(ixv[l] * _D, _D)
                pltpu.async_copy(tab_sh.at[pl.ds(off, _D)],
                                 gbs[b].at[pl.ds((j * 16 + l) * _D, _D)],
                                 sgs[b])

    def drain(b):
        pltpu.make_async_copy(feats_hbm.at[pl.ds(0, _CH)],
                              fbs[b], sfs[b]).wait()
        pltpu.make_async_copy(centers_hbm.at[pl.ds(0, _CH * _D)],
                              gbs[b], sgs[b]).wait()

    def compute_chunk(fbuf, gbuf, acc4):
        def row_body(r, acc4):
            a0, a1, a2, a3 = acc4
            g0 = r * _D
            for j in range(_D // 64):
                x0 = fbuf[r, pl.ds((4 * j + 0) * 16, 16)] - \
                    gbuf[pl.ds(g0 + (4 * j + 0) * 16, 16)]
                a0 = a0 + x0 * x0
                x1 = fbuf[r, pl.ds((4 * j + 1) * 16, 16)] - \
                    gbuf[pl.ds(g0 + (4 * j + 1) * 16, 16)]
                a1 = a1 + x1 * x1
                x2 = fbuf[r, pl.ds((4 * j + 2) * 16, 16)] - \
                    gbuf[pl.ds(g0 + (4 * j + 2) * 16, 16)]
                a2 = a2 + x2 * x2
                x3 = fbuf[r, pl.ds((4 * j + 3) * 16, 16)] - \
                    gbuf[pl.ds(g0 + (4 * j + 3) * 16, 16)]
                a3 = a3 + x3 * x3
            return (a0, a1, a2, a3)
        return plsc.parallel_loop(0, _CH, carry=acc4)(row_body)

    for b in range(_NBUF):
        issue(b, b)

    z = jnp.zeros((16,), jnp.float32)

    def group_body(gi, acc4):
        k0 = gi * _NBUF
        for b in range(_NBUF):
            k = k0 + b
            drain(b)
            acc4 = compute_chunk(fbs[b], gbs[b], acc4)

            @pl.when(k + _NBUF < _NCHUNK)
            def _():
                issue(k + _NBUF, b)
        return acc4

    acc4 = lax.fori_loop(0, _NCHUNK // _NBUF, group_body, (z, z, z, z))

    acc_v[...] = acc4[0] + acc4[1] + acc4[2] + acc4[3]
    pltpu.sync_copy(acc_v, out_hbm.at[wid])


_mesh = plsc.VectorSubcoreMesh(core_axis_name="c", subcore_axis_name="s")

_sc_partials = functools.partial(
    pl.kernel,
    out_type=jax.ShapeDtypeStruct((_NW, 16), jnp.float32),
    mesh=_mesh,
    scratch_types=[
        pltpu.VMEM((_NCHUNK, _CH), jnp.int32),
        pltpu.VMEM((_CH, _D), jnp.float32),
        pltpu.VMEM((_CH, _D), jnp.float32),
        pltpu.VMEM((_CH * _D,), jnp.float32),
        pltpu.VMEM((_CH * _D,), jnp.float32),
        pltpu.VMEM((16,), jnp.float32),
        pltpu.VMEM_SHARED((_C * _D,), jnp.float32),
        pltpu.SemaphoreType.DMA,
        pltpu.SemaphoreType.DMA,
        pltpu.SemaphoreType.DMA,
        pltpu.SemaphoreType.DMA,
        pltpu.SemaphoreType.DMA,
    ],
)(_body)


def _tc_body(lab_ref, feats_ref, cent_ref, out_ref):
    @pl.when(pl.program_id(0) == 0)
    def _():
        out_ref[...] = jnp.zeros_like(out_ref)

    lab = lab_ref[...]
    oh = (lab == lax.broadcasted_iota(jnp.int32, (_RB, _CPAD), 1)
          ).astype(jnp.bfloat16)
    g = lax.dot_general(oh, cent_ref[...], (((1,), (0,)), ((), ())),
                        preferred_element_type=jnp.float32)
    d = feats_ref[...] - g
    out_ref[...] += jnp.sum(d * d, axis=0, keepdims=True)


_tc_partial = pl.pallas_call(
    _tc_body,
    grid=(_BTC // _RB,),
    in_specs=[
        pl.BlockSpec((_RB, 1), lambda i: (_BSC // _RB + i, 0)),
        pl.BlockSpec((_RB, _D), lambda i: (_BSC // _RB + i, 0)),
        pl.BlockSpec((_CPAD, _D), lambda i: (0, 0)),
    ],
    out_specs=pl.BlockSpec((1, _D), lambda i: (0, 0)),
    out_shape=jax.ShapeDtypeStruct((1, _D), jnp.float32),
)


@jax.jit
def kernel(feats, labels, centers):
    labels = labels.astype(jnp.int32)
    sc_part = _sc_partials(feats, labels, centers.reshape(_C * _D))
    cent_bf = jnp.pad(centers, ((0, _CPAD - _C), (0, 0))
                      ).astype(jnp.bfloat16)
    tc_part = _tc_partial(labels.reshape(_B, 1), feats, cent_bf)
    total = jnp.sum(sc_part) + jnp.sum(tc_part)
    return total / jnp.float32(_B * _D)


# speedup vs baseline: 1.2603x; 1.2603x over previous
import functools

import jax
import jax.numpy as jnp
from jax import lax
from jax.experimental import pallas as pl
from jax.experimental.pallas import tpu as pltpu
from jax.experimental.pallas import tpu_sc as plsc

_B = 16384
_D = 512
_C = 1000
_CPAD = 1024

_BSC = 10240
_BTC = _B - _BSC
_RB = 256

_NC = 2
_NS = 16
_NW = _NC * _NS
_BPW = _BSC // _NW
_CH = 32
_NCHUNK = _BPW // _CH
_NBUF = 2


def _body(feats_hbm, labels_hbm, centers_hbm, out_hbm,
          idx_all, fb0, fb1, gb0, gb1, acc_v, tab_sh,
          sf0, sf1, sg0, sg1, sl):
    cid = lax.axis_index("c")
    sid = lax.axis_index("s")
    wid = sid * _NC + cid
    base = wid * _BPW

    fbs = (fb0, fb1)
    gbs = (gb0, gb1)
    sfs = (sf0, sf1)
    sgs = (sg0, sg1)

    @pl.when(sid < 8)
    def _():
        o = sid * (_C // 8) * _D
        pltpu.sync_copy(centers_hbm.at[pl.ds(o, (_C // 8) * _D)],
                        tab_sh.at[pl.ds(o, (_C // 8) * _D)])

    lcps = [
        pltpu.async_copy(labels_hbm.at[pl.ds(base + k * _CH, _CH)],
                         idx_all.at[k], sl)
        for k in range(_NCHUNK)
    ]
    for cp in lcps:
        cp.wait()
    plsc.subcore_barrier()

    def issue(k, b):
        pltpu.async_copy(feats_hbm.at[pl.ds(base + k * _CH, _CH)],
                         fbs[b], sfs[b])
        for j in range(_CH // 16):
            ixv = idx_all[k, pl.ds(j * 16, 16)]
            for l in range(16):
                off = pl.multiple_of(ixv[l] * _D, _D)
                pltpu.async_copy(tab_sh.at[pl.ds(off, _D)],
                                 gbs[b].at[pl.ds((j * 16 + l) * _D, _D)],
                                 sgs[b])

    def drain(b):
        pltpu.make_async_copy(feats_hbm.at[pl.ds(0, _CH)],
                              fbs[b], sfs[b]).wait()
        pltpu.make_async_copy(centers_hbm.at[pl.ds(0, _CH * _D)],
                              gbs[b], sgs[b]).wait()

    def compute_chunk(fbuf, gbuf, acc4):
        def row_body(r, acc4):
            a0, a1, a2, a3 = acc4
            g0 = r * _D
            for j in range(_D // 64):
                x0 = fbuf[r, pl.ds((4 * j + 0) * 16, 16)] - \
                    gbuf[pl.ds(g0 + (4 * j + 0) * 16, 16)]
                a0 = a0 + x0 * x0
                x1 = fbuf[r, pl.ds((4 * j + 1) * 16, 16)] - \
                    gbuf[pl.ds(g0 + (4 * j + 1) * 16, 16)]
                a1 = a1 + x1 * x1
                x2 = fbuf[r, pl.ds((4 * j + 2) * 16, 16)] - \
                    gbuf[pl.ds(g0 + (4 * j + 2) * 16, 16)]
                a2 = a2 + x2 * x2
                x3 = fbuf[r, pl.ds((4 * j + 3) * 16, 16)] - \
                    gbuf[pl.ds(g0 + (4 * j + 3) * 16, 16)]
                a3 = a3 + x3 * x3
            return (a0, a1, a2, a3)
        return plsc.parallel_loop(0, _CH, carry=acc4)(row_body)

    for b in range(_NBUF):
        issue(b, b)

    z = jnp.zeros((16,), jnp.float32)

    def group_body(gi, acc4):
        k0 = gi * _NBUF
        for b in range(_NBUF):
            k = k0 + b
            drain(b)
            acc4 = compute_chunk(fbs[b], gbs[b], acc4)

            @pl.when(k + _NBUF < _NCHUNK)
            def _():
                issue(k + _NBUF, b)
        return acc4

    acc4 = lax.fori_loop(0, _NCHUNK // _NBUF, group_body, (z, z, z, z))

    acc_v[...] = acc4[0] + acc4[1] + acc4[2] + acc4[3]
    pltpu.sync_copy(acc_v, out_hbm.at[wid])


_mesh = plsc.VectorSubcoreMesh(core_axis_name="c", subcore_axis_name="s")

_sc_partials = functools.partial(
    pl.kernel,
    out_type=jax.ShapeDtypeStruct((_NW, 16), jnp.float32),
    mesh=_mesh,
    scratch_types=[
        pltpu.VMEM((_NCHUNK, _CH), jnp.int32),
        pltpu.VMEM((_CH, _D), jnp.float32),
        pltpu.VMEM((_CH, _D), jnp.float32),
        pltpu.VMEM((_CH * _D,), jnp.float32),
        pltpu.VMEM((_CH * _D,), jnp.float32),
        pltpu.VMEM((16,), jnp.float32),
        pltpu.VMEM_SHARED((_C * _D,), jnp.float32),
        pltpu.SemaphoreType.DMA,
        pltpu.SemaphoreType.DMA,
        pltpu.SemaphoreType.DMA,
        pltpu.SemaphoreType.DMA,
        pltpu.SemaphoreType.DMA,
    ],
)(_body)


def _tc_body(lab_ref, feats_ref, cent_ref, out_ref):
    @pl.when(pl.program_id(0) == 0)
    def _():
        out_ref[...] = jnp.zeros_like(out_ref)

    lab = jnp.reshape(lab_ref[...], (_RB, 1))
    oh = (lab == lax.broadcasted_iota(jnp.int32, (_RB, _CPAD), 1)
          ).astype(jnp.bfloat16)
    g = lax.dot_general(oh, cent_ref[...], (((1,), (0,)), ((), ())),
                        preferred_element_type=jnp.float32)
    d = feats_ref[...] - g
    out_ref[...] += jnp.sum(d * d, axis=0, keepdims=True)


_tc_partial = pl.pallas_call(
    _tc_body,
    grid=(_BTC // _RB,),
    in_specs=[
        pl.BlockSpec((1, 1, _RB), lambda i: (_BSC // _RB + i, 0, 0)),
        pl.BlockSpec((_RB, _D), lambda i: (_BSC // _RB + i, 0)),
        pl.BlockSpec((_CPAD, _D), lambda i: (0, 0)),
    ],
    out_specs=pl.BlockSpec((1, _D), lambda i: (0, 0)),
    out_shape=jax.ShapeDtypeStruct((1, _D), jnp.float32),
)


@jax.jit
def kernel(feats, labels, centers):
    labels = labels.astype(jnp.int32)
    sc_part = _sc_partials(feats, labels, centers.reshape(_C * _D))
    cent_bf = jnp.pad(centers, ((0, _CPAD - _C), (0, 0))
                      ).astype(jnp.bfloat16)
    tc_part = _tc_partial(labels.reshape(_B // _RB, 1, _RB), feats, cent_bf)
    total = jnp.sum(sc_part) + jnp.sum(tc_part)
    return total / jnp.float32(_B * _D)
